# baseline (speedup 1.0000x reference)
import dataclasses
import functools

import jax
import jax.numpy as jnp
from jax import lax
from jax.experimental import pallas as pl
from jax.experimental.pallas import tpu as pltpu
from jax.experimental.pallas import tpu_sc as plsc

N = 10000
E = 320000
D = 128
EPS = 1e-5

NC = 2
NS = 16
NL = 16
NW = NC * NS
EPW = E // NW
K = 80
NCHUNK = EPW // K
RB = 80
NRB = N // RB
ZB = K
NZB = N // ZB


def _vector_mesh():
    return plsc.VectorSubcoreMesh(core_axis_name="c", subcore_axis_name="s")


def _no_layout_params():
    cp = pltpu.CompilerParams()
    if "needs_layout_passes" in pltpu.CompilerParams.__dataclass_fields__:
        cp = dataclasses.replace(cp, needs_layout_passes=False)
    return cp


def _sc_degree(ei):
    @functools.partial(
        pl.kernel,
        out_type=jax.ShapeDtypeStruct((NW, N), jnp.float32),
        mesh=_vector_mesh(),
        compiler_params=_no_layout_params(),
        scratch_types=[
            pltpu.VMEM((EPW,), jnp.int32),
            pltpu.VMEM((N,), jnp.float32),
        ],
    )
    def deg_kernel(ei_hbm, out_hbm, dst_v, hist_v):
        c = lax.axis_index("c")
        s = lax.axis_index("s")
        w = c * NS + s

        @pl.loop(0, N, step=NL)
        def _(i):
            hist_v[pl.ds(i, NL)] = jnp.zeros((NL,), jnp.float32)

        pltpu.sync_copy(ei_hbm.at[pl.ds(E + w * EPW, EPW)], dst_v)
        ones = jnp.ones((NL,), jnp.float32)

        @pl.loop(0, EPW, step=NL)
        def _(i):
            idx = dst_v[pl.ds(i, NL)]
            plsc.addupdate_scatter(hist_v, [idx], ones)

        pltpu.sync_copy(hist_v, out_hbm.at[w])

    return deg_kernel(ei)


def _sc_scatter(y, ei):
    K2 = 40
    NCH = EPW // K2
    NSL = 9
    GD = 7
    SD = NSL - GD

    @functools.partial(
        pl.kernel,
        out_type=jax.ShapeDtypeStruct((NC, N, D), jnp.float32),
        mesh=_vector_mesh(),
        scratch_types=(
            [pltpu.VMEM((K2,), jnp.int32) for _ in range(NSL)]
            + [pltpu.VMEM((K2,), jnp.int32) for _ in range(NSL)]
            + [pltpu.VMEM((K2, D), jnp.float32) for _ in range(NSL)]
            + [pltpu.VMEM_SHARED((N, D), jnp.float32)]
            + [pltpu.SemaphoreType.DMA for _ in range(2 * NSL + 2)]
        ),
    )
    def mp_kernel(ei_hbm, y_hbm, out_hbm, *scr):
        sidx = scr[0:NSL]
        didx = scr[NSL:2 * NSL]
        rows = scr[2 * NSL:3 * NSL]
        acc_sh = scr[3 * NSL]
        sg = scr[3 * NSL + 1:3 * NSL + 1 + NSL]
        ss = scr[3 * NSL + 1 + NSL:3 * NSL + 1 + 2 * NSL]
        ssrc = scr[3 * NSL + 1 + 2 * NSL]
        sdst = scr[3 * NSL + 2 + 2 * NSL]

        c = lax.axis_index("c")
        s = lax.axis_index("s")
        w = c * NS + s
        base = w * EPW

        def sidx_copy(ch, u):
            return pltpu.make_async_copy(
                ei_hbm.at[pl.ds(base + ch * K2, K2)], sidx[u], ssrc)

        def didx_copy(ch, u):
            return pltpu.make_async_copy(
                ei_hbm.at[pl.ds(E + base + ch * K2, K2)], didx[u], sdst)

        def g_copy(u):
            return pltpu.make_async_copy(y_hbm.at[sidx[u]], rows[u], sg[u])

        def s_wait(u):
            pltpu.make_async_copy(rows[u], acc_sh.at[didx[u]], ss[u]).wait()

        def s_start(u):
            pltpu.async_copy(rows[u], acc_sh.at[didx[u]], ss[u], add=True)

        for u in range(NSL):
            sidx_copy(u, u).start()
            didx_copy(u, u).start()

        @pl.loop(0, K2)
        def _(r):
            @pl.loop(0, D, step=NL)
            def _(j):
                rows[0][r, pl.ds(j, NL)] = jnp.zeros((NL,), jnp.float32)

        nz = (NCH - s + NS - 1) // NS

        @pl.loop(s, NCH, step=NS)
        def _(b):
            pltpu.make_async_copy(
                rows[0], acc_sh.at[pl.ds(b * K2, K2)], ss[0]).start()

        @pl.loop(0, nz)
        def _(b):
            pltpu.make_async_copy(
                rows[0], acc_sh.at[pl.ds(0, K2)], ss[0]).wait()

        plsc.subcore_barrier()

        for u in range(GD):
            sidx_copy(u, u).wait()
            g_copy(u).start()

        @pl.loop(0, 261, step=NSL)
        def _(i):
            for u in range(NSL):
                ch = i + u
                ug = (u + GD) % NSL

                @pl.when(ch < NCH)
                def _():
                    g_copy(u).wait()
                    didx_copy(ch, u).wait()
                    s_start(u)

                @pl.when(ch + NSL < NCH)
                def _():
                    sidx_copy(ch + NSL, u).start()

                @pl.when(jnp.logical_and(SD <= ch, ch - SD < NCH))
                def _():
                    s_wait(ug)

                @pl.when(jnp.logical_and(NSL <= ch + GD, ch + GD < NCH))
                def _():
                    didx_copy(ch + GD, ug).start()

                @pl.when(ch + GD < NCH)
                def _():
                    sidx_copy(ch + GD, ug).wait()
                    g_copy(ug).start()

        plsc.subcore_barrier()

        nd = (NRB - s + NS - 1) // NS

        @pl.loop(s, NRB, step=NS)
        def _(b):
            pltpu.make_async_copy(
                acc_sh.at[pl.ds(b * RB, RB)],
                out_hbm.at[c, pl.ds(b * RB, RB)], ss[0]).start()

        @pl.loop(0, nd)
        def _(b):
            pltpu.make_async_copy(
                acc_sh.at[pl.ds(0, RB)],
                out_hbm.at[c, pl.ds(0, RB)], ss[0]).wait()

    return mp_kernel(ei, y)


def _tc_matmul_scale(x, w, degp):
    def body(x_ref, w_ref, degp_ref, y_ref, dinv_ref):
        xw = jnp.dot(x_ref[...], w_ref[...],
                     preferred_element_type=jnp.float32)
        deg = jnp.sum(degp_ref[...], axis=0) + 1.0
        dinv = lax.rsqrt(jnp.maximum(deg, 1.0))[:, None]
        dinv_ref[...] = dinv
        y_ref[...] = dinv * xw

    return pl.pallas_call(
        body,
        out_shape=(jax.ShapeDtypeStruct((N, D), jnp.float32),
                   jax.ShapeDtypeStruct((N, 1), jnp.float32)))(x, w, degp)


def _tc_mid(s1, y1, dinv1, b1, gamma, beta, w2):
    def body(s_ref, y1_ref, dinv_ref, b1_ref, g_ref, bt_ref, w2_ref, y2_ref):
        dinv = dinv_ref[...]
        h = dinv * (s_ref[0] + s_ref[1] + y1_ref[...]) + b1_ref[...]
        mean = jnp.mean(h, axis=0, keepdims=True)
        cent = h - mean
        var = jnp.mean(cent * cent, axis=0, keepdims=True)
        hn = cent * lax.rsqrt(var + EPS) * g_ref[...] + bt_ref[...]
        hn = jnp.maximum(hn, 0.0)
        y2_ref[...] = dinv * jnp.dot(hn, w2_ref[...],
                                     preferred_element_type=jnp.float32)

    return pl.pallas_call(
        body, out_shape=jax.ShapeDtypeStruct((N, D), jnp.float32))(
            s1, y1, dinv1, b1, gamma, beta, w2)


def _tc_final(s2, y2, dinv1, b2):
    NB = 10
    R = N // NB

    def body(s_ref, y2_ref, dinv_ref, b2_ref, o_ref):
        h = dinv_ref[...] * (s_ref[0] + s_ref[1] + y2_ref[...]) + b2_ref[...]
        o_ref[...] = jnp.maximum(h, 0.0)

    return pl.pallas_call(
        body,
        grid=(NB,),
        in_specs=[pl.BlockSpec((NC, R, D), lambda i: (0, i, 0)),
                  pl.BlockSpec((R, D), lambda i: (i, 0)),
                  pl.BlockSpec((R, 1), lambda i: (i, 0)),
                  pl.BlockSpec((D,), lambda i: (0,))],
        out_specs=pl.BlockSpec((R, D), lambda i: (i, 0)),
        out_shape=jax.ShapeDtypeStruct((N, D), jnp.float32))(
            s2, y2, dinv1, b2)


def kernel(x, edge_index, W1, b1, bn_gamma, bn_beta, W2, b2):
    ei = edge_index.astype(jnp.int32).reshape(2 * E)

    degp = _sc_degree(ei)
    y1, dinv1 = _tc_matmul_scale(x, W1, degp)
    s1 = _sc_scatter(y1, ei)
    y2 = _tc_mid(s1, y1, dinv1, b1, bn_gamma, bn_beta, W2)
    s2 = _sc_scatter(y2, ei)
    return _tc_final(s2, y2, dinv1, b2)


# speedup vs baseline: 1.0058x; 1.0058x over previous
import dataclasses
import functools

import jax
import jax.numpy as jnp
from jax import lax
from jax.experimental import pallas as pl
from jax.experimental.pallas import tpu as pltpu
from jax.experimental.pallas import tpu_sc as plsc

N = 10000
E = 320000
D = 128
EPS = 1e-5

NC = 2
NS = 16
NL = 16
NW = NC * NS
EPW = E // NW
K = 80
NCHUNK = EPW // K
RB = 80
NRB = N // RB
ZB = K
NZB = N // ZB


def _vector_mesh():
    return plsc.VectorSubcoreMesh(core_axis_name="c", subcore_axis_name="s")


def _no_layout_params():
    cp = pltpu.CompilerParams()
    if "needs_layout_passes" in pltpu.CompilerParams.__dataclass_fields__:
        cp = dataclasses.replace(cp, needs_layout_passes=False)
    return cp


def _sc_degree(ei):
    @functools.partial(
        pl.kernel,
        out_type=jax.ShapeDtypeStruct((NW, N), jnp.float32),
        mesh=_vector_mesh(),
        compiler_params=_no_layout_params(),
        scratch_types=[
            pltpu.VMEM((EPW,), jnp.int32),
            pltpu.VMEM((N,), jnp.float32),
        ],
    )
    def deg_kernel(ei_hbm, out_hbm, dst_v, hist_v):
        c = lax.axis_index("c")
        s = lax.axis_index("s")
        w = c * NS + s

        @pl.loop(0, N, step=NL)
        def _(i):
            hist_v[pl.ds(i, NL)] = jnp.zeros((NL,), jnp.float32)

        pltpu.sync_copy(ei_hbm.at[pl.ds(E + w * EPW, EPW)], dst_v)
        ones = jnp.ones((NL,), jnp.float32)

        @pl.loop(0, EPW, step=NL)
        def _(i):
            idx = dst_v[pl.ds(i, NL)]
            plsc.addupdate_scatter(hist_v, [idx], ones)

        pltpu.sync_copy(hist_v, out_hbm.at[w])

    return deg_kernel(ei)


def _sc_scatter(y, ei):
    K2 = 40
    NCH = EPW // K2
    NSL = 9
    GD = 7
    SD = NSL - GD

    @functools.partial(
        pl.kernel,
        out_type=jax.ShapeDtypeStruct((NC, N, D), jnp.float32),
        mesh=_vector_mesh(),
        scratch_types=(
            [pltpu.VMEM((K2,), jnp.int32) for _ in range(NSL)]
            + [pltpu.VMEM((K2,), jnp.int32) for _ in range(NSL)]
            + [pltpu.VMEM((K2, D), jnp.float32) for _ in range(NSL)]
            + [pltpu.VMEM_SHARED((N, D), jnp.float32)]
            + [pltpu.SemaphoreType.DMA for _ in range(2 * NSL + 2)]
        ),
    )
    def mp_kernel(ei_hbm, y_hbm, out_hbm, *scr):
        sidx = scr[0:NSL]
        didx = scr[NSL:2 * NSL]
        rows = scr[2 * NSL:3 * NSL]
        acc_sh = scr[3 * NSL]
        sg = scr[3 * NSL + 1:3 * NSL + 1 + NSL]
        ss = scr[3 * NSL + 1 + NSL:3 * NSL + 1 + 2 * NSL]
        ssrc = scr[3 * NSL + 1 + 2 * NSL]
        sdst = scr[3 * NSL + 2 + 2 * NSL]

        c = lax.axis_index("c")
        s = lax.axis_index("s")
        w = c * NS + s
        base = w * EPW

        def sidx_copy(ch, u):
            return pltpu.make_async_copy(
                ei_hbm.at[pl.ds(base + ch * K2, K2)], sidx[u], ssrc)

        def didx_copy(ch, u):
            return pltpu.make_async_copy(
                ei_hbm.at[pl.ds(E + base + ch * K2, K2)], didx[u], sdst)

        def g_copy(u):
            return pltpu.make_async_copy(y_hbm.at[sidx[u]], rows[u], sg[u])

        def s_wait(u):
            pltpu.make_async_copy(rows[u], acc_sh.at[didx[u]], ss[u]).wait()

        def s_start(u):
            pltpu.async_copy(rows[u], acc_sh.at[didx[u]], ss[u], add=True)

        for u in range(NSL):
            sidx_copy(u, u).start()
            didx_copy(u, u).start()

        @pl.loop(0, K2)
        def _(r):
            @pl.loop(0, D, step=NL)
            def _(j):
                rows[0][r, pl.ds(j, NL)] = jnp.zeros((NL,), jnp.float32)

        nz = (NCH - s + NS - 1) // NS

        @pl.loop(s, NCH, step=NS)
        def _(b):
            pltpu.make_async_copy(
                rows[0], acc_sh.at[pl.ds(b * K2, K2)], ss[0]).start()

        @pl.loop(0, nz)
        def _(b):
            pltpu.make_async_copy(
                rows[0], acc_sh.at[pl.ds(0, K2)], ss[0]).wait()

        plsc.subcore_barrier()

        for u in range(GD):
            sidx_copy(u, u).wait()
            g_copy(u).start()

        @pl.loop(0, 261, step=NSL)
        def _(i):
            for u in range(NSL):
                ch = i + u
                ug = (u + GD) % NSL

                @pl.when(ch < NCH)
                def _():
                    g_copy(u).wait()
                    didx_copy(ch, u).wait()
                    s_start(u)

                @pl.when(ch + NSL < NCH)
                def _():
                    sidx_copy(ch + NSL, u).start()

                @pl.when(jnp.logical_and(SD <= ch, ch - SD < NCH))
                def _():
                    s_wait(ug)

                @pl.when(jnp.logical_and(NSL <= ch + GD, ch + GD < NCH))
                def _():
                    didx_copy(ch + GD, ug).start()

                @pl.when(ch + GD < NCH)
                def _():
                    sidx_copy(ch + GD, ug).wait()
                    g_copy(ug).start()

        plsc.subcore_barrier()

        nd = (NRB - s + NS - 1) // NS

        @pl.loop(s, NRB, step=NS)
        def _(b):
            pltpu.make_async_copy(
                acc_sh.at[pl.ds(b * RB, RB)],
                out_hbm.at[c, pl.ds(b * RB, RB)], ss[0]).start()

        @pl.loop(0, nd)
        def _(b):
            pltpu.make_async_copy(
                acc_sh.at[pl.ds(0, RB)],
                out_hbm.at[c, pl.ds(0, RB)], ss[0]).wait()

    return mp_kernel(ei, y)


def _tc_matmul_scale(x, w, degp):
    def body(x_ref, w_ref, degp_ref, y_ref, dinv_ref):
        xw = jnp.dot(x_ref[...], w_ref[...],
                     preferred_element_type=jnp.float32)
        deg = jnp.sum(degp_ref[...], axis=0) + 1.0
        dinv = lax.rsqrt(jnp.maximum(deg, 1.0))[:, None]
        dinv_ref[...] = dinv
        y_ref[...] = dinv * xw

    return pl.pallas_call(
        body,
        out_shape=(jax.ShapeDtypeStruct((N, D), jnp.float32),
                   jax.ShapeDtypeStruct((N, 1), jnp.float32)))(x, w, degp)


def _tc_mid(s1, y1, dinv1, b1, gamma, beta, w2):
    def body(s_ref, y1_ref, dinv_ref, b1_ref, g_ref, bt_ref, w2_ref, y2_ref):
        dinv = dinv_ref[...]
        h = dinv * (s_ref[0] + s_ref[1] + y1_ref[...]) + b1_ref[...]
        mean = jnp.mean(h, axis=0, keepdims=True)
        cent = h - mean
        var = jnp.mean(cent * cent, axis=0, keepdims=True)
        hn = cent * lax.rsqrt(var + EPS) * g_ref[...] + bt_ref[...]
        hn = jnp.maximum(hn, 0.0)
        y2_ref[...] = dinv * jnp.dot(hn, w2_ref[...],
                                     preferred_element_type=jnp.float32)

    return pl.pallas_call(
        body, out_shape=jax.ShapeDtypeStruct((N, D), jnp.float32))(
            s1, y1, dinv1, b1, gamma, beta, w2)


def _tc_final(s2, y2, dinv1, b2):
    def body(s_ref, y2_ref, dinv_ref, b2_ref, o_ref):
        h = dinv_ref[...] * (s_ref[0] + s_ref[1] + y2_ref[...]) + b2_ref[...]
        o_ref[...] = jnp.maximum(h, 0.0)

    return pl.pallas_call(
        body, out_shape=jax.ShapeDtypeStruct((N, D), jnp.float32))(
            s2, y2, dinv1, b2)


def kernel(x, edge_index, W1, b1, bn_gamma, bn_beta, W2, b2):
    ei = edge_index.astype(jnp.int32).reshape(2 * E)

    degp = _sc_degree(ei)
    y1, dinv1 = _tc_matmul_scale(x, W1, degp)
    s1 = _sc_scatter(y1, ei)
    y2 = _tc_mid(s1, y1, dinv1, b1, bn_gamma, bn_beta, W2)
    s2 = _sc_scatter(y2, ei)
    return _tc_final(s2, y2, dinv1, b2)
